# Initial kernel scaffold; baseline (speedup 1.0000x reference)
import jax
import jax.numpy as jnp
from jax.experimental import pallas as pl


def kernel(positions, hidden_states, Wq_swa, Wk_swa, Wv_swa, Wq_hsa, Wk_hsa, Wv_hsa, W_lmk, W_gate, Wo, q_norm_w, k_norm_w, lmk_norm_w):
    raise NotImplementedError("write your pallas kernel here")


# speedup vs baseline: 1.4823x; 1.4823x over previous
import functools
import math

import jax
import jax.numpy as jnp
from jax.experimental import pallas as pl

B, S, D = 1, 2048, 2048
DH = 128
HQ_SWA, HK_SWA = 12, 6
HQ_HSA, HK_HSA = 4, 2
WINDOW = 512
CHUNK = 64
TOPK = 8
THETA = 1e6
EPS = 1e-6
SCALE = DH ** -0.5

NCOLS = 38 * DH


def _proj_kern(x_ref, w_ref, o_ref):
    o_ref[...] = jnp.dot(x_ref[...], w_ref[...],
                         preferred_element_type=jnp.float32)


def _projection(x, wcat):
    bn = 256
    return pl.pallas_call(
        _proj_kern,
        grid=(NCOLS // bn,),
        in_specs=[
            pl.BlockSpec((S, D), lambda n: (0, 0)),
            pl.BlockSpec((D, bn), lambda n: (0, n)),
        ],
        out_specs=pl.BlockSpec((S, bn), lambda n: (0, n)),
        out_shape=jax.ShapeDtypeStruct((S, NCOLS), jnp.float32),
    )(x, wcat)


def _rms(y, w):
    v = jnp.mean(y * y, axis=-1, keepdims=True)
    return y * jax.lax.rsqrt(v + EPS) * w[None, None, :]


def _rope(y, cos, sin):
    half = DH // 2
    x1 = y[..., :half]
    x2 = y[..., half:]
    c = cos[:, None, :]
    s = sin[:, None, :]
    return jnp.concatenate([x1 * c - x2 * s, x2 * c + x1 * s], axis=-1)


def _prep_kern(p_ref, qn_ref, kn_ref, ln_ref,
               qswa_ref, kswa_ref, qhsa_ref, khsa_ref, lmkq_ref, lmkk_ref,
               *, bs):
    sblk = pl.program_id(0)
    rows = sblk * bs + jax.lax.broadcasted_iota(jnp.int32, (bs, 1), 0)
    half = DH // 2
    fi = jax.lax.broadcasted_iota(jnp.int32, (1, half), 1).astype(jnp.float32)
    freqs = jnp.exp(fi * (-math.log(THETA) / half))
    ang = rows.astype(jnp.float32) * freqs
    cos = jnp.cos(ang)
    sin = jnp.sin(ang)

    qw = qn_ref[...]
    kw = kn_ref[...]
    lw = ln_ref[...]

    def grab(c0, nh):
        return p_ref[:, c0 * DH:(c0 + nh) * DH].reshape(bs, nh, DH)

    qswa = _rope(_rms(grab(0, HQ_SWA), qw), cos, sin)
    qswa_ref[...] = qswa.reshape(bs, HQ_SWA * DH)
    kswa = _rope(_rms(grab(12, HK_SWA), kw), cos, sin)
    kswa_ref[...] = kswa.reshape(bs, HK_SWA * DH)
    qhsa = _rope(_rms(grab(24, HQ_HSA), qw), cos, sin)
    qhsa_ref[...] = qhsa.reshape(bs, HQ_HSA * DH)

    khsa_n = _rms(grab(28, HK_HSA), kw)
    khsa = _rope(khsa_n, cos, sin)
    khsa_ref[...] = khsa.reshape(bs, HK_HSA * DH)

    lmkq = _rms(grab(32, HK_HSA), lw)
    lmkq_ref[...] = lmkq.reshape(bs, HK_HSA * DH)

    nck = bs // CHUNK
    lmkk_ref[0] = khsa_n.reshape(nck, CHUNK, HK_HSA * DH).mean(axis=1)


def _prep(P, q_norm_w, k_norm_w, lmk_norm_w):
    bs = 256
    nck = bs // CHUNK
    grid = (S // bs,)
    kern = functools.partial(_prep_kern, bs=bs)
    outs = pl.pallas_call(
        kern,
        grid=grid,
        in_specs=[
            pl.BlockSpec((bs, NCOLS), lambda s: (s, 0)),
            pl.BlockSpec((DH,), lambda s: (0,)),
            pl.BlockSpec((DH,), lambda s: (0,)),
            pl.BlockSpec((DH,), lambda s: (0,)),
        ],
        out_specs=[
            pl.BlockSpec((bs, HQ_SWA * DH), lambda s: (s, 0)),
            pl.BlockSpec((bs, HK_SWA * DH), lambda s: (s, 0)),
            pl.BlockSpec((bs, HQ_HSA * DH), lambda s: (s, 0)),
            pl.BlockSpec((bs, HK_HSA * DH), lambda s: (s, 0)),
            pl.BlockSpec((bs, HK_HSA * DH), lambda s: (s, 0)),
            pl.BlockSpec((1, nck, HK_HSA * DH), lambda s: (s, 0, 0)),
        ],
        out_shape=[
            jax.ShapeDtypeStruct((S, HQ_SWA * DH), jnp.float32),
            jax.ShapeDtypeStruct((S, HK_SWA * DH), jnp.float32),
            jax.ShapeDtypeStruct((S, HQ_HSA * DH), jnp.float32),
            jax.ShapeDtypeStruct((S, HK_HSA * DH), jnp.float32),
            jax.ShapeDtypeStruct((S, HK_HSA * DH), jnp.float32),
            jax.ShapeDtypeStruct((S // bs, nck, HK_HSA * DH), jnp.float32),
        ],
    )(P, q_norm_w, k_norm_w, lmk_norm_w)
    qswa, kswa, qhsa, khsa, lmkq, lmkk_p = outs
    lmkk = lmkk_p.reshape(S // CHUNK, HK_HSA, DH)
    return qswa, kswa, qhsa, khsa, lmkq, lmkk


def _swa_kern(q_ref, k_ref, v_ref, o_ref, *, bq, kspan):
    qb = pl.program_id(1)
    q0 = qb * bq
    k0 = jnp.maximum(q0 - WINDOW, 0)

    q = q_ref[...]
    k = k_ref[pl.ds(k0, kspan), :]
    v = v_ref[pl.ds(k0, kspan), :]

    s = jax.lax.dot_general(q, k, (((1,), (1,)), ((), ()))) * SCALE
    i = q0 + jax.lax.broadcasted_iota(jnp.int32, (bq, kspan), 0)
    j = k0 + jax.lax.broadcasted_iota(jnp.int32, (bq, kspan), 1)
    mask = (j <= i) & ((i - j) < WINDOW)
    s = jnp.where(mask, s, -1e9)
    m = jnp.max(s, axis=-1, keepdims=True)
    e = jnp.exp(s - m)
    p = e / jnp.sum(e, axis=-1, keepdims=True)
    o_ref[...] = jnp.dot(p, v, preferred_element_type=jnp.float32)


def _swa(qswa, kswa, P):
    bq = 256
    kspan = WINDOW + bq
    kern = functools.partial(_swa_kern, bq=bq, kspan=kspan)
    return pl.pallas_call(
        kern,
        grid=(HQ_SWA, S // bq),
        in_specs=[
            pl.BlockSpec((bq, DH), lambda h, qb: (qb, h)),
            pl.BlockSpec((S, DH), lambda h, qb: (0, h // 2)),
            pl.BlockSpec((S, DH), lambda h, qb: (0, 18 + h // 2)),
        ],
        out_specs=pl.BlockSpec((bq, DH), lambda h, qb: (qb, h)),
        out_shape=jax.ShapeDtypeStruct((S, HQ_SWA * DH), jnp.float32),
    )(qswa, kswa, P)


def _hsa_kern(q_ref, lq_ref, lk_ref, k_ref, v_ref, g_ref, o_ref, *, bq):
    qb = pl.program_id(1)
    q0 = qb * bq
    nC = S // CHUNK

    lq = lq_ref[...]
    lk = lk_ref[0]
    lsc = jax.lax.dot_general(lq, lk, (((1,), (1,)), ((), ()))) * SCALE
    rows = q0 + jax.lax.broadcasted_iota(jnp.int32, (bq, nC), 0)
    cidx = jax.lax.broadcasted_iota(jnp.int32, (bq, nC), 1)
    cmask = (cidx * CHUNK) <= rows
    work = jnp.where(cmask, lsc, -1e9)
    cidx_f = cidx.astype(jnp.float32)
    sel = jnp.zeros((bq, nC), jnp.bool_)
    for _ in range(TOPK):
        mx = jnp.max(work, axis=-1, keepdims=True)
        eq = work == mx
        fidx = jnp.min(jnp.where(eq, cidx_f, 1e9), axis=-1, keepdims=True)
        first = cidx_f == fidx
        sel = sel | first
        work = jnp.where(first, -jnp.inf, work)
    cur = cidx == (rows // CHUNK)
    sel = (sel | cur) & cmask

    c_row = jax.lax.broadcasted_iota(jnp.int32, (nC, S), 0)
    j_col = jax.lax.broadcasted_iota(jnp.int32, (nC, S), 1)
    expand = ((j_col // CHUNK) == c_row).astype(jnp.float32)
    tok_f = jnp.dot(sel.astype(jnp.float32), expand,
                    preferred_element_type=jnp.float32)

    q = q_ref[...]
    k = k_ref[...]
    v = v_ref[...]
    s = jax.lax.dot_general(q, k, (((1,), (1,)), ((), ()))) * SCALE
    i = q0 + jax.lax.broadcasted_iota(jnp.int32, (bq, S), 0)
    j = jax.lax.broadcasted_iota(jnp.int32, (bq, S), 1)
    s = jnp.where((tok_f > 0.5) & (j <= i), s, -1e9)
    m = jnp.max(s, axis=-1, keepdims=True)
    e = jnp.exp(s - m)
    p = e / jnp.sum(e, axis=-1, keepdims=True)
    o = jnp.dot(p, v, preferred_element_type=jnp.float32)
    gate = jax.nn.sigmoid(g_ref[...])
    o_ref[...] = o * gate


def _hsa(qhsa, khsa, lmkq, lmkk, P):
    bq = 256
    nC = S // CHUNK
    kern = functools.partial(_hsa_kern, bq=bq)
    lmkk3 = lmkk.transpose(1, 0, 2)
    return pl.pallas_call(
        kern,
        grid=(HQ_HSA, S // bq),
        in_specs=[
            pl.BlockSpec((bq, DH), lambda h, qb: (qb, h)),
            pl.BlockSpec((bq, DH), lambda h, qb: (qb, h // 2)),
            pl.BlockSpec((1, nC, DH), lambda h, qb: (h // 2, 0, 0)),
            pl.BlockSpec((S, DH), lambda h, qb: (0, h // 2)),
            pl.BlockSpec((S, DH), lambda h, qb: (0, 30 + h // 2)),
            pl.BlockSpec((bq, DH), lambda h, qb: (qb, 34 + h)),
        ],
        out_specs=pl.BlockSpec((bq, DH), lambda h, qb: (qb, h)),
        out_shape=jax.ShapeDtypeStruct((S, HQ_HSA * DH), jnp.float32),
    )(qhsa, lmkq, lmkk3, khsa, P, P)


def _out_kern(a_ref, b_ref, w1_ref, w2_ref, o_ref):
    acc = jnp.dot(a_ref[...], w1_ref[...], preferred_element_type=jnp.float32)
    acc += jnp.dot(b_ref[...], w2_ref[...], preferred_element_type=jnp.float32)
    o_ref[...] = acc


def _outproj(swa_o, hsa_o, Wo):
    bn = 512
    wa = Wo[:HQ_SWA * DH]
    wb = Wo[HQ_SWA * DH:]
    return pl.pallas_call(
        _out_kern,
        grid=(D // bn,),
        in_specs=[
            pl.BlockSpec((S, HQ_SWA * DH), lambda n: (0, 0)),
            pl.BlockSpec((S, HQ_HSA * DH), lambda n: (0, 0)),
            pl.BlockSpec((HQ_SWA * DH, bn), lambda n: (0, n)),
            pl.BlockSpec((HQ_HSA * DH, bn), lambda n: (0, n)),
        ],
        out_specs=pl.BlockSpec((S, bn), lambda n: (0, n)),
        out_shape=jax.ShapeDtypeStruct((S, D), jnp.float32),
    )(swa_o, hsa_o, wa, wb)


@jax.jit
def kernel(positions, hidden_states, Wq_swa, Wk_swa, Wv_swa, Wq_hsa, Wk_hsa,
           Wv_hsa, W_lmk, W_gate, Wo, q_norm_w, k_norm_w, lmk_norm_w):
    wcat = jnp.concatenate(
        [Wq_swa, Wk_swa, Wv_swa, Wq_hsa, Wk_hsa, Wv_hsa, W_lmk, W_gate],
        axis=1)
    P = _projection(hidden_states, wcat)
    qswa, kswa, qhsa, khsa, lmkq, lmkk = _prep(
        P, q_norm_w, k_norm_w, lmk_norm_w)
    swa_o = _swa(qswa, kswa, P)
    hsa_o = _hsa(qhsa, khsa, lmkq, lmkk, P)
    return _outproj(swa_o, hsa_o, Wo)


# speedup vs baseline: 1.6093x; 1.0857x over previous
import functools
import math

import jax
import jax.numpy as jnp
from jax.experimental import pallas as pl

B, S, D = 1, 2048, 2048
DH = 128
HQ_SWA, HK_SWA = 12, 6
HQ_HSA, HK_HSA = 4, 2
WINDOW = 512
CHUNK = 64
TOPK = 8
THETA = 1e6
EPS = 1e-6
SCALE = DH ** -0.5

NCOLS = 38 * DH


def _proj_kern(x_ref, w_ref, o_ref):
    o_ref[...] = jnp.dot(x_ref[...], w_ref[...],
                         preferred_element_type=jnp.float32)


def _projection(x, wcat):
    bn = 256
    return pl.pallas_call(
        _proj_kern,
        grid=(NCOLS // bn,),
        in_specs=[
            pl.BlockSpec((S, D), lambda n: (0, 0)),
            pl.BlockSpec((D, bn), lambda n: (0, n)),
        ],
        out_specs=pl.BlockSpec((S, bn), lambda n: (0, n)),
        out_shape=jax.ShapeDtypeStruct((S, NCOLS), jnp.float32),
    )(x, wcat)


def _rms(y, w):
    v = jnp.mean(y * y, axis=-1, keepdims=True)
    return y * jax.lax.rsqrt(v + EPS) * w[None, None, :]


def _rope(y, cos, sin):
    half = DH // 2
    x1 = y[..., :half]
    x2 = y[..., half:]
    c = cos[:, None, :]
    s = sin[:, None, :]
    return jnp.concatenate([x1 * c - x2 * s, x2 * c + x1 * s], axis=-1)


def _prep_kern(p_ref, qn_ref, kn_ref, ln_ref,
               qswa_ref, kswa_ref, qhsa_ref, khsa_ref, lmkq_ref, lmkk_ref,
               *, bs):
    sblk = pl.program_id(0)
    rows = sblk * bs + jax.lax.broadcasted_iota(jnp.int32, (bs, 1), 0)
    half = DH // 2
    fi = jax.lax.broadcasted_iota(jnp.int32, (1, half), 1).astype(jnp.float32)
    freqs = jnp.exp(fi * (-math.log(THETA) / half))
    ang = rows.astype(jnp.float32) * freqs
    cos = jnp.cos(ang)
    sin = jnp.sin(ang)

    qw = qn_ref[...]
    kw = kn_ref[...]
    lw = ln_ref[...]

    def grab(c0, nh):
        return p_ref[:, c0 * DH:(c0 + nh) * DH].reshape(bs, nh, DH)

    qswa = _rope(_rms(grab(0, HQ_SWA), qw), cos, sin)
    qswa_ref[...] = qswa.reshape(bs, HQ_SWA * DH)
    kswa = _rope(_rms(grab(12, HK_SWA), kw), cos, sin)
    kswa_ref[...] = kswa.reshape(bs, HK_SWA * DH)
    qhsa = _rope(_rms(grab(24, HQ_HSA), qw), cos, sin)
    qhsa_ref[...] = qhsa.reshape(bs, HQ_HSA * DH)

    khsa_n = _rms(grab(28, HK_HSA), kw)
    khsa = _rope(khsa_n, cos, sin)
    khsa_ref[...] = khsa.reshape(bs, HK_HSA * DH)

    lmkq = _rms(grab(32, HK_HSA), lw)
    lmkq_ref[...] = lmkq.reshape(bs, HK_HSA * DH)

    nck = bs // CHUNK
    lmkk_ref[0] = khsa_n.reshape(nck, CHUNK, HK_HSA * DH).mean(axis=1)


def _prep(P, q_norm_w, k_norm_w, lmk_norm_w):
    bs = 256
    nck = bs // CHUNK
    grid = (S // bs,)
    kern = functools.partial(_prep_kern, bs=bs)
    outs = pl.pallas_call(
        kern,
        grid=grid,
        in_specs=[
            pl.BlockSpec((bs, NCOLS), lambda s: (s, 0)),
            pl.BlockSpec((DH,), lambda s: (0,)),
            pl.BlockSpec((DH,), lambda s: (0,)),
            pl.BlockSpec((DH,), lambda s: (0,)),
        ],
        out_specs=[
            pl.BlockSpec((bs, HQ_SWA * DH), lambda s: (s, 0)),
            pl.BlockSpec((bs, HK_SWA * DH), lambda s: (s, 0)),
            pl.BlockSpec((bs, HQ_HSA * DH), lambda s: (s, 0)),
            pl.BlockSpec((bs, HK_HSA * DH), lambda s: (s, 0)),
            pl.BlockSpec((bs, HK_HSA * DH), lambda s: (s, 0)),
            pl.BlockSpec((1, nck, HK_HSA * DH), lambda s: (s, 0, 0)),
        ],
        out_shape=[
            jax.ShapeDtypeStruct((S, HQ_SWA * DH), jnp.float32),
            jax.ShapeDtypeStruct((S, HK_SWA * DH), jnp.float32),
            jax.ShapeDtypeStruct((S, HQ_HSA * DH), jnp.float32),
            jax.ShapeDtypeStruct((S, HK_HSA * DH), jnp.float32),
            jax.ShapeDtypeStruct((S, HK_HSA * DH), jnp.float32),
            jax.ShapeDtypeStruct((S // bs, nck, HK_HSA * DH), jnp.float32),
        ],
    )(P, q_norm_w, k_norm_w, lmk_norm_w)
    qswa, kswa, qhsa, khsa, lmkq, lmkk_p = outs
    lmkk = lmkk_p.reshape(S // CHUNK, HK_HSA, DH)
    return qswa, kswa, qhsa, khsa, lmkq, lmkk


def _swa_kern(q_ref, k_ref, v_ref, m_ref, o_ref, *, bq, kspan):
    qb = pl.program_id(1)
    q0 = qb * bq
    k0 = pl.multiple_of(jnp.maximum(q0 - WINDOW, 0), 256)

    q = q_ref[...]
    k = k_ref[pl.ds(k0, kspan), :]
    v = v_ref[pl.ds(k0, kspan), :]

    s = jax.lax.dot_general(q, k, (((1,), (1,)), ((), ()))) * SCALE
    e = jnp.exp(s + m_ref[0])
    den = jnp.sum(e, axis=-1, keepdims=True)
    pv = jnp.dot(e, v, preferred_element_type=jnp.float32)
    o_ref[...] = pv * (1.0 / den)


def _swa(qswa, kswa, P, maskadd):
    bq = 256
    kspan = WINDOW + bq
    kern = functools.partial(_swa_kern, bq=bq, kspan=kspan)
    return pl.pallas_call(
        kern,
        grid=(HQ_SWA, S // bq),
        in_specs=[
            pl.BlockSpec((bq, DH), lambda h, qb: (qb, h)),
            pl.BlockSpec((S, DH), lambda h, qb: (0, h // 2)),
            pl.BlockSpec((S, DH), lambda h, qb: (0, 18 + h // 2)),
            pl.BlockSpec((1, bq, kspan),
                         lambda h, qb: (jnp.minimum(qb, 2), 0, 0)),
        ],
        out_specs=pl.BlockSpec((bq, DH), lambda h, qb: (qb, h)),
        out_shape=jax.ShapeDtypeStruct((S, HQ_SWA * DH), jnp.float32),
    )(qswa, kswa, P, maskadd)


def _hsa_kern(q_ref, lq_ref, lk_ref, k_ref, v_ref, g_ref, o_ref, *, bq,
              qb0, kw):
    qb = qb0 + pl.program_id(1)
    q0 = qb * bq
    nC = S // CHUNK

    lq = lq_ref[...]
    lk = lk_ref[0]
    lsc = jax.lax.dot_general(lq, lk, (((1,), (1,)), ((), ()))) * SCALE
    rows = q0 + jax.lax.broadcasted_iota(jnp.int32, (bq, nC), 0)
    cidx = jax.lax.broadcasted_iota(jnp.int32, (bq, nC), 1)
    cmask = (cidx * CHUNK) <= rows
    work = jnp.where(cmask, lsc, -1e9)
    cidx_f = cidx.astype(jnp.float32)
    sel = jnp.zeros((bq, nC), jnp.bool_)
    for _ in range(TOPK):
        mx = jnp.max(work, axis=-1, keepdims=True)
        eq = work == mx
        fidx = jnp.min(jnp.where(eq, cidx_f, 1e9), axis=-1, keepdims=True)
        first = cidx_f == fidx
        sel = sel | first
        work = jnp.where(first, -jnp.inf, work)
    cur = cidx == (rows // CHUNK)
    sel = (sel | cur) & cmask

    c_row = jax.lax.broadcasted_iota(jnp.int32, (nC, kw), 0)
    j_col = jax.lax.broadcasted_iota(jnp.int32, (nC, kw), 1)
    expand = ((j_col // CHUNK) == c_row).astype(jnp.float32)
    tok_f = jnp.dot(sel.astype(jnp.float32), expand,
                    preferred_element_type=jnp.float32)

    q = q_ref[...]
    k = k_ref[...]
    v = v_ref[...]
    s = jax.lax.dot_general(q, k, (((1,), (1,)), ((), ()))) * SCALE
    i = q0 + jax.lax.broadcasted_iota(jnp.int32, (bq, kw), 0)
    j = jax.lax.broadcasted_iota(jnp.int32, (bq, kw), 1)
    s = jnp.where((tok_f > 0.5) & (j <= i), s, -1e9)
    e = jnp.exp(s)
    den = jnp.sum(e, axis=-1, keepdims=True)
    pv = jnp.dot(e, v, preferred_element_type=jnp.float32)
    gate = jax.nn.sigmoid(g_ref[...])
    o_ref[...] = pv * (1.0 / den) * gate


def _hsa_part(qhsa, khsa, lmkq, lmkk3, P, qb0, nqb, kw):
    bq = 256
    nC = S // CHUNK
    kern = functools.partial(_hsa_kern, bq=bq, qb0=qb0, kw=kw)
    return pl.pallas_call(
        kern,
        grid=(HQ_HSA, nqb),
        in_specs=[
            pl.BlockSpec((bq, DH), lambda h, qb: (qb0 + qb, h)),
            pl.BlockSpec((bq, DH), lambda h, qb: (qb0 + qb, h // 2)),
            pl.BlockSpec((1, nC, DH), lambda h, qb: (h // 2, 0, 0)),
            pl.BlockSpec((kw, DH), lambda h, qb: (0, h // 2)),
            pl.BlockSpec((kw, DH), lambda h, qb: (0, 30 + h // 2)),
            pl.BlockSpec((bq, DH), lambda h, qb: (qb0 + qb, 34 + h)),
        ],
        out_specs=pl.BlockSpec((bq, DH), lambda h, qb: (qb0 + qb, h)),
        out_shape=jax.ShapeDtypeStruct((S, HQ_HSA * DH), jnp.float32),
    )(qhsa, lmkq, lmkk3, khsa, P, P)


def _hsa(qhsa, khsa, lmkq, lmkk, P):
    lmkk3 = lmkk.transpose(1, 0, 2)
    lo = _hsa_part(qhsa, khsa, lmkq, lmkk3, P, 0, 4, 1024)
    hi = _hsa_part(qhsa, khsa, lmkq, lmkk3, P, 4, 4, S)
    return jnp.concatenate([lo[:S // 2], hi[S // 2:]], axis=0)


def _out_kern(a_ref, b_ref, w1_ref, w2_ref, o_ref):
    acc = jnp.dot(a_ref[...], w1_ref[...], preferred_element_type=jnp.float32)
    acc += jnp.dot(b_ref[...], w2_ref[...], preferred_element_type=jnp.float32)
    o_ref[...] = acc


def _outproj(swa_o, hsa_o, Wo):
    bn = 512
    wa = Wo[:HQ_SWA * DH]
    wb = Wo[HQ_SWA * DH:]
    return pl.pallas_call(
        _out_kern,
        grid=(D // bn,),
        in_specs=[
            pl.BlockSpec((S, HQ_SWA * DH), lambda n: (0, 0)),
            pl.BlockSpec((S, HQ_HSA * DH), lambda n: (0, 0)),
            pl.BlockSpec((HQ_SWA * DH, bn), lambda n: (0, n)),
            pl.BlockSpec((HQ_HSA * DH, bn), lambda n: (0, n)),
        ],
        out_specs=pl.BlockSpec((S, bn), lambda n: (0, n)),
        out_shape=jax.ShapeDtypeStruct((S, D), jnp.float32),
    )(swa_o, hsa_o, wa, wb)


@jax.jit
def kernel(positions, hidden_states, Wq_swa, Wk_swa, Wv_swa, Wq_hsa, Wk_hsa,
           Wv_hsa, W_lmk, W_gate, Wo, q_norm_w, k_norm_w, lmk_norm_w):
    wcat = jnp.concatenate(
        [Wq_swa, Wk_swa, Wv_swa, Wq_hsa, Wk_hsa, Wv_hsa, W_lmk, W_gate],
        axis=1)
    P = _projection(hidden_states, wcat)
    qswa, kswa, qhsa, khsa, lmkq, lmkk = _prep(
        P, q_norm_w, k_norm_w, lmk_norm_w)
    bq, kspan = 256, WINDOW + 256
    r = jnp.arange(bq)[:, None]
    t = jnp.arange(kspan)[None, :]
    masks = []
    for mm in range(3):
        q0 = mm * bq
        k0 = max(q0 - WINDOW, 0)
        i = q0 + r
        j = k0 + t
        ok = (j <= i) & ((i - j) < WINDOW)
        masks.append(jnp.where(ok, 0.0, -1e9))
    maskadd = jnp.stack(masks)
    swa_o = _swa(qswa, kswa, P, maskadd)
    hsa_o = _hsa(qhsa, khsa, lmkq, lmkk, P)
    return _outproj(swa_o, hsa_o, Wo)


# speedup vs baseline: 1.7790x; 1.1054x over previous
import functools
import math

import jax
import jax.numpy as jnp
from jax.experimental import pallas as pl
from jax.experimental.pallas import tpu as pltpu

B, S, D = 1, 2048, 2048
DH = 128
HQ_SWA, HK_SWA = 12, 6
HQ_HSA, HK_HSA = 4, 2
WINDOW = 512
CHUNK = 64
TOPK = 8
THETA = 1e6
EPS = 1e-6
SCALE = DH ** -0.5

NCOLS = 38 * DH


_WOFFS = (0, 6, 9, 12, 14, 15, 16, 17)
_WNB = (6, 3, 3, 2, 1, 1, 1, 2)


def _proj_kern(x_ref, *refs):
    w_refs = refs[:8]
    o_ref = refs[8]
    wbuf_ref = refs[9]
    n = pl.program_id(0)
    for jj in range(8):
        lo = _WOFFS[jj]
        hi = _WOFFS[jj + 1] if jj < 7 else NCOLS // 256

        @pl.when((n >= lo) & (n < hi))
        def _(jj=jj):
            wbuf_ref[...] = w_refs[jj][...]

    o_ref[...] = jnp.dot(x_ref[...], wbuf_ref[...],
                         preferred_element_type=jnp.float32)


def _projection(x, ws):
    bn = 256

    def wmap(off, nb):
        return lambda n: (0, jnp.clip(n - off, 0, nb - 1))

    return pl.pallas_call(
        _proj_kern,
        grid=(NCOLS // bn,),
        in_specs=[pl.BlockSpec((S, D), lambda n: (0, 0))] + [
            pl.BlockSpec((D, bn), wmap(_WOFFS[jj], _WNB[jj]))
            for jj in range(8)
        ],
        out_specs=pl.BlockSpec((S, bn), lambda n: (0, n)),
        out_shape=jax.ShapeDtypeStruct((S, NCOLS), jnp.float32),
        scratch_shapes=[pltpu.VMEM((D, bn), jnp.float32)],
        compiler_params=pltpu.CompilerParams(
            vmem_limit_bytes=96 * 1024 * 1024),
    )(x, *ws)


def _rms2(y, w):
    v = jnp.mean(y * y, axis=-1, keepdims=True)
    return y * jax.lax.rsqrt(v + EPS) * w


def _rope2(y, cos2, sgnsin):
    return y * cos2 + pltpu.roll(y, DH // 2, 1) * sgnsin


def _prep_kern(p_ref, qn_ref, kn_ref, ln_ref, cos_ref, sin_ref,
               qswa_ref, kswa_ref, qhsa_ref, khsa_ref, lmkq_ref, lmkk_ref,
               *, bs):
    cos2 = cos_ref[...]
    sgnsin = sin_ref[...]
    qw = qn_ref[...]
    kw = kn_ref[...]
    lw = ln_ref[...]

    def col(c):
        return p_ref[:, c * DH:(c + 1) * DH]

    for h in range(HQ_SWA):
        z = _rms2(col(0 + h), qw)
        qswa_ref[:, h * DH:(h + 1) * DH] = _rope2(z, cos2, sgnsin)
    for h in range(HK_SWA):
        z = _rms2(col(12 + h), kw)
        kswa_ref[:, h * DH:(h + 1) * DH] = _rope2(z, cos2, sgnsin)
    for h in range(HQ_HSA):
        z = _rms2(col(24 + h), qw)
        qhsa_ref[:, h * DH:(h + 1) * DH] = _rope2(z, cos2, sgnsin)

    khn = []
    for h in range(HK_HSA):
        z = _rms2(col(28 + h), kw)
        khn.append(z)
        khsa_ref[:, h * DH:(h + 1) * DH] = _rope2(z, cos2, sgnsin)
    for h in range(HK_HSA):
        lmkq_ref[:, h * DH:(h + 1) * DH] = _rms2(col(32 + h), lw)

    nck = bs // CHUNK
    khsa_n = jnp.concatenate(khn, axis=1)
    lmkk_ref[0] = khsa_n.reshape(nck, CHUNK, HK_HSA * DH).mean(axis=1)


def _prep(P, q_norm_w, k_norm_w, lmk_norm_w, cos2, sgnsin):
    bs = 256
    nck = bs // CHUNK
    grid = (S // bs,)
    kern = functools.partial(_prep_kern, bs=bs)
    outs = pl.pallas_call(
        kern,
        grid=grid,
        in_specs=[
            pl.BlockSpec((bs, NCOLS), lambda s: (s, 0)),
            pl.BlockSpec((1, DH), lambda s: (0, 0)),
            pl.BlockSpec((1, DH), lambda s: (0, 0)),
            pl.BlockSpec((1, DH), lambda s: (0, 0)),
            pl.BlockSpec((bs, DH), lambda s: (s, 0)),
            pl.BlockSpec((bs, DH), lambda s: (s, 0)),
        ],
        out_specs=[
            pl.BlockSpec((bs, HQ_SWA * DH), lambda s: (s, 0)),
            pl.BlockSpec((bs, HK_SWA * DH), lambda s: (s, 0)),
            pl.BlockSpec((bs, HQ_HSA * DH), lambda s: (s, 0)),
            pl.BlockSpec((bs, HK_HSA * DH), lambda s: (s, 0)),
            pl.BlockSpec((bs, HK_HSA * DH), lambda s: (s, 0)),
            pl.BlockSpec((1, nck, HK_HSA * DH), lambda s: (s, 0, 0)),
        ],
        out_shape=[
            jax.ShapeDtypeStruct((S, HQ_SWA * DH), jnp.float32),
            jax.ShapeDtypeStruct((S, HK_SWA * DH), jnp.float32),
            jax.ShapeDtypeStruct((S, HQ_HSA * DH), jnp.float32),
            jax.ShapeDtypeStruct((S, HK_HSA * DH), jnp.float32),
            jax.ShapeDtypeStruct((S, HK_HSA * DH), jnp.float32),
            jax.ShapeDtypeStruct((S // bs, nck, HK_HSA * DH), jnp.float32),
        ],
    )(P, q_norm_w.reshape(1, DH), k_norm_w.reshape(1, DH),
      lmk_norm_w.reshape(1, DH), cos2, sgnsin)
    qswa, kswa, qhsa, khsa, lmkq, lmkk_p = outs
    lmkk = lmkk_p.reshape(S // CHUNK, HK_HSA, DH)
    return qswa, kswa, qhsa, khsa, lmkq, lmkk


def _swa_kern(q_ref, k_ref, v_ref, m_ref, o_ref, *, bq, kspan):
    qb = pl.program_id(1)
    q0 = qb * bq
    k0 = pl.multiple_of(jnp.maximum(q0 - WINDOW, 0), 256)

    q = q_ref[...]
    k = k_ref[pl.ds(k0, kspan), :]
    v = v_ref[pl.ds(k0, kspan), :]

    s = jax.lax.dot_general(q, k, (((1,), (1,)), ((), ()))) * SCALE
    e = jnp.exp(s + m_ref[0])
    den = jnp.sum(e, axis=-1, keepdims=True)
    pv = jnp.dot(e, v, preferred_element_type=jnp.float32)
    o_ref[...] = pv * (1.0 / den)


def _swa(qswa, kswa, P, maskadd):
    bq = 256
    kspan = WINDOW + bq
    kern = functools.partial(_swa_kern, bq=bq, kspan=kspan)
    return pl.pallas_call(
        kern,
        grid=(HQ_SWA, S // bq),
        in_specs=[
            pl.BlockSpec((bq, DH), lambda h, qb: (qb, h)),
            pl.BlockSpec((S, DH), lambda h, qb: (0, h // 2)),
            pl.BlockSpec((S, DH), lambda h, qb: (0, 18 + h // 2)),
            pl.BlockSpec((1, bq, kspan),
                         lambda h, qb: (jnp.minimum(qb, 2), 0, 0)),
        ],
        out_specs=pl.BlockSpec((bq, DH), lambda h, qb: (qb, h)),
        out_shape=jax.ShapeDtypeStruct((S, HQ_SWA * DH), jnp.float32),
    )(qswa, kswa, P, maskadd)


def _hsa_kern(q_ref, lq_ref, lk_ref, k_ref, v_ref, g_ref, o_ref, *, bq,
              qb0, kw):
    qb = qb0 + pl.program_id(1)
    q0 = qb * bq
    nC = S // CHUNK

    lq = lq_ref[...]
    lk = lk_ref[0]
    lsc = jax.lax.dot_general(lq, lk, (((1,), (1,)), ((), ()))) * SCALE
    rows = q0 + jax.lax.broadcasted_iota(jnp.int32, (bq, nC), 0)
    cidx = jax.lax.broadcasted_iota(jnp.int32, (bq, nC), 1)
    cmask = (cidx * CHUNK) <= rows
    work = jnp.where(cmask, lsc, -1e9)
    cidx_f = cidx.astype(jnp.float32)
    sel = jnp.zeros((bq, nC), jnp.bool_)
    for _ in range(TOPK):
        mx = jnp.max(work, axis=-1, keepdims=True)
        eq = work == mx
        fidx = jnp.min(jnp.where(eq, cidx_f, 1e9), axis=-1, keepdims=True)
        first = cidx_f == fidx
        sel = sel | first
        work = jnp.where(first, -jnp.inf, work)
    cur = cidx == (rows // CHUNK)
    sel = (sel | cur) & cmask

    c_row = jax.lax.broadcasted_iota(jnp.int32, (nC, kw), 0)
    j_col = jax.lax.broadcasted_iota(jnp.int32, (nC, kw), 1)
    expand = ((j_col // CHUNK) == c_row).astype(jnp.float32)
    tok_f = jnp.dot(sel.astype(jnp.float32), expand,
                    preferred_element_type=jnp.float32)

    q = q_ref[...]
    k = k_ref[...]
    v = v_ref[...]
    s = jax.lax.dot_general(q, k, (((1,), (1,)), ((), ()))) * SCALE
    i = q0 + jax.lax.broadcasted_iota(jnp.int32, (bq, kw), 0)
    j = jax.lax.broadcasted_iota(jnp.int32, (bq, kw), 1)
    s = jnp.where((tok_f > 0.5) & (j <= i), s, -1e9)
    e = jnp.exp(s)
    den = jnp.sum(e, axis=-1, keepdims=True)
    pv = jnp.dot(e, v, preferred_element_type=jnp.float32)
    gate = jax.nn.sigmoid(g_ref[...])
    o_ref[...] = pv * (1.0 / den) * gate


def _hsa_part(qhsa, khsa, lmkq, lmkk3, P, qb0, nqb, kw):
    bq = 256
    nC = S // CHUNK
    kern = functools.partial(_hsa_kern, bq=bq, qb0=qb0, kw=kw)
    return pl.pallas_call(
        kern,
        grid=(HQ_HSA, nqb),
        in_specs=[
            pl.BlockSpec((bq, DH), lambda h, qb: (qb0 + qb, h)),
            pl.BlockSpec((bq, DH), lambda h, qb: (qb0 + qb, h // 2)),
            pl.BlockSpec((1, nC, DH), lambda h, qb: (h // 2, 0, 0)),
            pl.BlockSpec((kw, DH), lambda h, qb: (0, h // 2)),
            pl.BlockSpec((kw, DH), lambda h, qb: (0, 30 + h // 2)),
            pl.BlockSpec((bq, DH), lambda h, qb: (qb0 + qb, 34 + h)),
        ],
        out_specs=pl.BlockSpec((bq, DH), lambda h, qb: (qb0 + qb, h)),
        out_shape=jax.ShapeDtypeStruct((S, HQ_HSA * DH), jnp.float32),
    )(qhsa, lmkq, lmkk3, khsa, P, P)


def _hsa(qhsa, khsa, lmkq, lmkk, P):
    lmkk3 = lmkk.transpose(1, 0, 2)
    lo = _hsa_part(qhsa, khsa, lmkq, lmkk3, P, 0, 4, 1024)
    hi = _hsa_part(qhsa, khsa, lmkq, lmkk3, P, 4, 4, S)
    return jnp.concatenate([lo[:S // 2], hi[S // 2:]], axis=0)


def _out_kern(a_ref, b_ref, w1_ref, w2_ref, o_ref):
    acc = jnp.dot(a_ref[...], w1_ref[...], preferred_element_type=jnp.float32)
    acc += jnp.dot(b_ref[...], w2_ref[...], preferred_element_type=jnp.float32)
    o_ref[...] = acc


def _outproj(swa_o, hsa_o, Wo):
    bn = 512
    wa = Wo[:HQ_SWA * DH]
    wb = Wo[HQ_SWA * DH:]
    return pl.pallas_call(
        _out_kern,
        grid=(D // bn,),
        in_specs=[
            pl.BlockSpec((S, HQ_SWA * DH), lambda n: (0, 0)),
            pl.BlockSpec((S, HQ_HSA * DH), lambda n: (0, 0)),
            pl.BlockSpec((HQ_SWA * DH, bn), lambda n: (0, n)),
            pl.BlockSpec((HQ_HSA * DH, bn), lambda n: (0, n)),
        ],
        out_specs=pl.BlockSpec((S, bn), lambda n: (0, n)),
        out_shape=jax.ShapeDtypeStruct((S, D), jnp.float32),
    )(swa_o, hsa_o, wa, wb)


@jax.jit
def kernel(positions, hidden_states, Wq_swa, Wk_swa, Wv_swa, Wq_hsa, Wk_hsa,
           Wv_hsa, W_lmk, W_gate, Wo, q_norm_w, k_norm_w, lmk_norm_w):
    P = _projection(hidden_states,
                    (Wq_swa, Wk_swa, Wv_swa, Wq_hsa, Wk_hsa, Wv_hsa,
                     W_lmk, W_gate))
    pos = positions.astype(jnp.float32)
    half = DH // 2
    freqs = 1.0 / (THETA ** (jnp.arange(half, dtype=jnp.float32) / half))
    ang = pos[:, None] * freqs[None, :]
    cosv = jnp.cos(ang)
    sinv = jnp.sin(ang)
    cos2 = jnp.concatenate([cosv, cosv], axis=1)
    sgnsin = jnp.concatenate([-sinv, sinv], axis=1)
    qswa, kswa, qhsa, khsa, lmkq, lmkk = _prep(
        P, q_norm_w, k_norm_w, lmk_norm_w, cos2, sgnsin)
    bq, kspan = 256, WINDOW + 256
    r = jnp.arange(bq)[:, None]
    t = jnp.arange(kspan)[None, :]
    masks = []
    for mm in range(3):
        q0 = mm * bq
        k0 = max(q0 - WINDOW, 0)
        i = q0 + r
        j = k0 + t
        ok = (j <= i) & ((i - j) < WINDOW)
        masks.append(jnp.where(ok, 0.0, -1e9))
    maskadd = jnp.stack(masks)
    swa_o = _swa(qswa, kswa, P, maskadd)
    hsa_o = _hsa(qhsa, khsa, lmkq, lmkk, P)
    return _outproj(swa_o, hsa_o, Wo)


# speedup vs baseline: 1.7850x; 1.0033x over previous
import functools
import math

import jax
import jax.numpy as jnp
from jax.experimental import pallas as pl
from jax.experimental.pallas import tpu as pltpu

B, S, D = 1, 2048, 2048
DH = 128
HQ_SWA, HK_SWA = 12, 6
HQ_HSA, HK_HSA = 4, 2
WINDOW = 512
CHUNK = 64
TOPK = 8
THETA = 1e6
EPS = 1e-6
SCALE = DH ** -0.5

NCOLS = 38 * DH


_WOFFS = (0, 6, 9, 12, 14, 15, 16, 17)
_WNB = (6, 3, 3, 2, 1, 1, 1, 2)


def _proj_kern(x_ref, *refs):
    w_refs = refs[:8]
    o_ref = refs[8]
    wbuf_ref = refs[9]
    n = pl.program_id(0)
    for jj in range(8):
        lo = _WOFFS[jj]
        hi = _WOFFS[jj + 1] if jj < 7 else NCOLS // 256

        @pl.when((n >= lo) & (n < hi))
        def _(jj=jj):
            wbuf_ref[...] = w_refs[jj][...]

    o_ref[...] = jnp.dot(x_ref[...], wbuf_ref[...],
                         preferred_element_type=jnp.float32)


def _projection(x, ws):
    bn = 256

    def wmap(off, nb):
        return lambda n: (0, jnp.clip(n - off, 0, nb - 1))

    return pl.pallas_call(
        _proj_kern,
        grid=(NCOLS // bn,),
        in_specs=[pl.BlockSpec((S, D), lambda n: (0, 0))] + [
            pl.BlockSpec((D, bn), wmap(_WOFFS[jj], _WNB[jj]))
            for jj in range(8)
        ],
        out_specs=pl.BlockSpec((S, bn), lambda n: (0, n)),
        out_shape=jax.ShapeDtypeStruct((S, NCOLS), jnp.float32),
        scratch_shapes=[pltpu.VMEM((D, bn), jnp.float32)],
        compiler_params=pltpu.CompilerParams(
            vmem_limit_bytes=96 * 1024 * 1024),
    )(x, *ws)


def _rms2(y, w):
    v = jnp.mean(y * y, axis=-1, keepdims=True)
    return y * jax.lax.rsqrt(v + EPS) * w


def _rope2(y, cos2, sgnsin):
    return y * cos2 + pltpu.roll(y, DH // 2, 1) * sgnsin


def _prep_kern(p_ref, qn_ref, kn_ref, ln_ref, cos_ref, sin_ref,
               qswa_ref, kswa_ref, qhsa_ref, khsa_ref, lmkq_ref, lmkk_ref,
               *, bs):
    cos2 = cos_ref[...]
    sgnsin = sin_ref[...]
    qw = qn_ref[...]
    kw = kn_ref[...]
    lw = ln_ref[...]

    def col(c):
        return p_ref[:, c * DH:(c + 1) * DH]

    for h in range(HQ_SWA):
        z = _rms2(col(0 + h), qw)
        qswa_ref[:, h * DH:(h + 1) * DH] = _rope2(z, cos2, sgnsin)
    for h in range(HK_SWA):
        z = _rms2(col(12 + h), kw)
        kswa_ref[:, h * DH:(h + 1) * DH] = _rope2(z, cos2, sgnsin)
    for h in range(HQ_HSA):
        z = _rms2(col(24 + h), qw)
        qhsa_ref[:, h * DH:(h + 1) * DH] = _rope2(z, cos2, sgnsin)

    khn = []
    for h in range(HK_HSA):
        z = _rms2(col(28 + h), kw)
        khn.append(z)
        khsa_ref[:, h * DH:(h + 1) * DH] = _rope2(z, cos2, sgnsin)
    for h in range(HK_HSA):
        lmkq_ref[:, h * DH:(h + 1) * DH] = _rms2(col(32 + h), lw)

    nck = bs // CHUNK
    khsa_n = jnp.concatenate(khn, axis=1)
    lmkk_ref[0] = khsa_n.reshape(nck, CHUNK, HK_HSA * DH).mean(axis=1)


def _prep(P, q_norm_w, k_norm_w, lmk_norm_w, cos2, sgnsin):
    bs = 256
    nck = bs // CHUNK
    grid = (S // bs,)
    kern = functools.partial(_prep_kern, bs=bs)
    outs = pl.pallas_call(
        kern,
        grid=grid,
        in_specs=[
            pl.BlockSpec((bs, NCOLS), lambda s: (s, 0)),
            pl.BlockSpec((1, DH), lambda s: (0, 0)),
            pl.BlockSpec((1, DH), lambda s: (0, 0)),
            pl.BlockSpec((1, DH), lambda s: (0, 0)),
            pl.BlockSpec((bs, DH), lambda s: (s, 0)),
            pl.BlockSpec((bs, DH), lambda s: (s, 0)),
        ],
        out_specs=[
            pl.BlockSpec((bs, HQ_SWA * DH), lambda s: (s, 0)),
            pl.BlockSpec((bs, HK_SWA * DH), lambda s: (s, 0)),
            pl.BlockSpec((bs, HQ_HSA * DH), lambda s: (s, 0)),
            pl.BlockSpec((bs, HK_HSA * DH), lambda s: (s, 0)),
            pl.BlockSpec((bs, HK_HSA * DH), lambda s: (s, 0)),
            pl.BlockSpec((1, nck, HK_HSA * DH), lambda s: (s, 0, 0)),
        ],
        out_shape=[
            jax.ShapeDtypeStruct((S, HQ_SWA * DH), jnp.float32),
            jax.ShapeDtypeStruct((S, HK_SWA * DH), jnp.float32),
            jax.ShapeDtypeStruct((S, HQ_HSA * DH), jnp.float32),
            jax.ShapeDtypeStruct((S, HK_HSA * DH), jnp.float32),
            jax.ShapeDtypeStruct((S, HK_HSA * DH), jnp.float32),
            jax.ShapeDtypeStruct((S // bs, nck, HK_HSA * DH), jnp.float32),
        ],
    )(P, q_norm_w.reshape(1, DH), k_norm_w.reshape(1, DH),
      lmk_norm_w.reshape(1, DH), cos2, sgnsin)
    qswa, kswa, qhsa, khsa, lmkq, lmkk_p = outs
    lmkk = lmkk_p.reshape(S // CHUNK, HK_HSA, DH)
    return qswa, kswa, qhsa, khsa, lmkq, lmkk


def _swa_kern(q_ref, k_ref, v_ref, m_ref, o_ref, *, bq, kspan):
    qb = pl.program_id(1)
    q0 = qb * bq
    k0 = pl.multiple_of(jnp.maximum(q0 - WINDOW, 0), 256)

    q = q_ref[...] * SCALE
    k = k_ref[pl.ds(k0, kspan), :]
    v = v_ref[pl.ds(k0, kspan), :]

    s = jax.lax.dot_general(q, k, (((1,), (1,)), ((), ())))
    e = jnp.exp(s + m_ref[0])
    den = jnp.sum(e, axis=-1, keepdims=True)
    pv = jnp.dot(e, v, preferred_element_type=jnp.float32)
    o_ref[...] = pv * (1.0 / den)


def _swa(qswa, kswa, P, maskadd):
    bq = 256
    kspan = WINDOW + bq
    kern = functools.partial(_swa_kern, bq=bq, kspan=kspan)
    return pl.pallas_call(
        kern,
        grid=(HQ_SWA, S // bq),
        in_specs=[
            pl.BlockSpec((bq, DH), lambda h, qb: (qb, h)),
            pl.BlockSpec((S, DH), lambda h, qb: (0, h // 2)),
            pl.BlockSpec((S, DH), lambda h, qb: (0, 18 + h // 2)),
            pl.BlockSpec((1, bq, kspan),
                         lambda h, qb: (jnp.minimum(qb, 2), 0, 0)),
        ],
        out_specs=pl.BlockSpec((bq, DH), lambda h, qb: (qb, h)),
        out_shape=jax.ShapeDtypeStruct((S, HQ_SWA * DH), jnp.float32),
    )(qswa, kswa, P, maskadd)


def _hsa_kern(q_ref, lq_ref, lk_ref, k_ref, v_ref, g_ref, o_ref, *, bq,
              qb0, kw):
    qb = qb0 + pl.program_id(1)
    q0 = qb * bq
    nC = S // CHUNK

    lq = lq_ref[...]
    lk = lk_ref[0]
    lsc = jax.lax.dot_general(lq, lk, (((1,), (1,)), ((), ()))) * SCALE
    rows = q0 + jax.lax.broadcasted_iota(jnp.int32, (bq, nC), 0)
    cidx = jax.lax.broadcasted_iota(jnp.int32, (bq, nC), 1)
    cmask = (cidx * CHUNK) <= rows
    work = jnp.where(cmask, lsc, -1e9)
    cidx_f = cidx.astype(jnp.float32)
    sel = jnp.zeros((bq, nC), jnp.bool_)
    for _ in range(TOPK):
        mx = jnp.max(work, axis=-1, keepdims=True)
        eq = work == mx
        fidx = jnp.min(jnp.where(eq, cidx_f, 1e9), axis=-1, keepdims=True)
        first = cidx_f == fidx
        sel = sel | first
        work = jnp.where(first, -jnp.inf, work)
    cur = cidx == (rows // CHUNK)
    sel = (sel | cur) & cmask

    c_row = jax.lax.broadcasted_iota(jnp.int32, (nC, kw), 0)
    j_col = jax.lax.broadcasted_iota(jnp.int32, (nC, kw), 1)
    expand = ((j_col // CHUNK) == c_row).astype(jnp.float32)
    tok_f = jnp.dot(sel.astype(jnp.float32), expand,
                    preferred_element_type=jnp.float32)

    q = q_ref[...] * SCALE
    k = k_ref[...]
    v = v_ref[...]
    s = jax.lax.dot_general(q, k, (((1,), (1,)), ((), ())))
    i = q0 + jax.lax.broadcasted_iota(jnp.int32, (bq, kw), 0)
    j = jax.lax.broadcasted_iota(jnp.int32, (bq, kw), 1)
    s = jnp.where((tok_f > 0.5) & (j <= i), s, -1e9)
    e = jnp.exp(s)
    den = jnp.sum(e, axis=-1, keepdims=True)
    pv = jnp.dot(e, v, preferred_element_type=jnp.float32)
    gate = jax.nn.sigmoid(g_ref[...])
    o_ref[...] = pv * (1.0 / den) * gate


def _hsa_part(qhsa, khsa, lmkq, lmkk3, P, qb0, nqb, kw):
    bq = 256
    nC = S // CHUNK
    kern = functools.partial(_hsa_kern, bq=bq, qb0=qb0, kw=kw)
    return pl.pallas_call(
        kern,
        grid=(HQ_HSA, nqb),
        in_specs=[
            pl.BlockSpec((bq, DH), lambda h, qb: (qb0 + qb, h)),
            pl.BlockSpec((bq, DH), lambda h, qb: (qb0 + qb, h // 2)),
            pl.BlockSpec((1, nC, DH), lambda h, qb: (h // 2, 0, 0)),
            pl.BlockSpec((kw, DH), lambda h, qb: (0, h // 2)),
            pl.BlockSpec((kw, DH), lambda h, qb: (0, 30 + h // 2)),
            pl.BlockSpec((bq, DH), lambda h, qb: (qb0 + qb, 34 + h)),
        ],
        out_specs=pl.BlockSpec((bq, DH), lambda h, qb: (qb0 + qb, h)),
        out_shape=jax.ShapeDtypeStruct((S, HQ_HSA * DH), jnp.float32),
    )(qhsa, lmkq, lmkk3, khsa, P, P)


def _hsa(qhsa, khsa, lmkq, lmkk, P):
    lmkk3 = lmkk.transpose(1, 0, 2)
    lo = _hsa_part(qhsa, khsa, lmkq, lmkk3, P, 0, 4, 1024)
    hi = _hsa_part(qhsa, khsa, lmkq, lmkk3, P, 4, 4, S)
    return jnp.concatenate([lo[:S // 2], hi[S // 2:]], axis=0)


def _out_kern(a_ref, b_ref, w1_ref, w2_ref, o_ref):
    acc = jnp.dot(a_ref[...], w1_ref[...], preferred_element_type=jnp.float32)
    acc += jnp.dot(b_ref[...], w2_ref[...], preferred_element_type=jnp.float32)
    o_ref[...] = acc


def _outproj(swa_o, hsa_o, Wo):
    bn = 512
    wa = Wo[:HQ_SWA * DH]
    wb = Wo[HQ_SWA * DH:]
    return pl.pallas_call(
        _out_kern,
        grid=(D // bn,),
        in_specs=[
            pl.BlockSpec((S, HQ_SWA * DH), lambda n: (0, 0)),
            pl.BlockSpec((S, HQ_HSA * DH), lambda n: (0, 0)),
            pl.BlockSpec((HQ_SWA * DH, bn), lambda n: (0, n)),
            pl.BlockSpec((HQ_HSA * DH, bn), lambda n: (0, n)),
        ],
        out_specs=pl.BlockSpec((S, bn), lambda n: (0, n)),
        out_shape=jax.ShapeDtypeStruct((S, D), jnp.float32),
    )(swa_o, hsa_o, wa, wb)


@jax.jit
def kernel(positions, hidden_states, Wq_swa, Wk_swa, Wv_swa, Wq_hsa, Wk_hsa,
           Wv_hsa, W_lmk, W_gate, Wo, q_norm_w, k_norm_w, lmk_norm_w):
    P = _projection(hidden_states,
                    (Wq_swa, Wk_swa, Wv_swa, Wq_hsa, Wk_hsa, Wv_hsa,
                     W_lmk, W_gate))
    pos = positions.astype(jnp.float32)
    half = DH // 2
    freqs = 1.0 / (THETA ** (jnp.arange(half, dtype=jnp.float32) / half))
    ang = pos[:, None] * freqs[None, :]
    cosv = jnp.cos(ang)
    sinv = jnp.sin(ang)
    cos2 = jnp.concatenate([cosv, cosv], axis=1)
    sgnsin = jnp.concatenate([-sinv, sinv], axis=1)
    qswa, kswa, qhsa, khsa, lmkq, lmkk = _prep(
        P, q_norm_w, k_norm_w, lmk_norm_w, cos2, sgnsin)
    bq, kspan = 256, WINDOW + 256
    r = jnp.arange(bq)[:, None]
    t = jnp.arange(kspan)[None, :]
    masks = []
    for mm in range(3):
        q0 = mm * bq
        k0 = max(q0 - WINDOW, 0)
        i = q0 + r
        j = k0 + t
        ok = (j <= i) & ((i - j) < WINDOW)
        masks.append(jnp.where(ok, 0.0, -1e9))
    maskadd = jnp.stack(masks)
    swa_o = _swa(qswa, kswa, P, maskadd)
    hsa_o = _hsa(qhsa, khsa, lmkq, lmkk, P)
    return _outproj(swa_o, hsa_o, Wo)
